# baseline (speedup 1.0000x reference)
import functools

import jax
import jax.numpy as jnp
from jax.experimental import pallas as pl
from jax.experimental.pallas import tpu as pltpu

_TOKEN_OFF = 4000
_TIME_ROWS = 64


def _gather_tile_kernel(TILE, ids_ref, tcol_ref, tab_ref, ttab_ref, pe_ref,
                        out_ref):
    base = pl.program_id(0) * TILE
    for mi in range(TILE):
        out_ref[mi] = tab_ref[ids_ref[base + mi], 0]
    lane = jax.lax.broadcasted_iota(jnp.int32, (TILE, _TIME_ROWS), 1)
    t_oh = (tcol_ref[...] == lane).astype(jnp.float32)
    tmm = jnp.dot(t_oh, ttab_ref[...], preferred_element_type=jnp.float32)
    out_ref[...] = out_ref[...] + (tmm + pe_ref[...])


def kernel(x, time, fused_table, pe_scaled):
    B, S = x.shape
    v_pad, d_model = fused_table.shape

    ids = x.astype(jnp.int32).reshape(B * S)
    tcol = time.astype(jnp.int32).reshape(B * S, 1)
    tab3d = fused_table.reshape(v_pad, 1, d_model)
    ttab = fused_table[_TOKEN_OFF:_TOKEN_OFF + _TIME_ROWS]
    pe = pe_scaled[:S]

    tile = 256
    while S % tile:
        tile //= 2
    n_s = S // tile
    grid = (B * n_s,)

    body = functools.partial(_gather_tile_kernel, tile)

    m_total = B * S
    bytes_accessed = (2 * m_total * 4
                      + v_pad * d_model * 4
                      + S * d_model * 4
                      + m_total * d_model * 4)
    cost = pl.CostEstimate(flops=3 * m_total * d_model, transcendentals=0,
                           bytes_accessed=bytes_accessed)

    out = pl.pallas_call(
        body,
        out_shape=jax.ShapeDtypeStruct((B * S, d_model), jnp.float32),
        grid=grid,
        in_specs=[
            pl.BlockSpec(memory_space=pltpu.SMEM),
            pl.BlockSpec((tile, 1), lambda i: (i, 0)),
            pl.BlockSpec((v_pad, 1, d_model), lambda i: (0, 0, 0)),
            pl.BlockSpec((_TIME_ROWS, d_model), lambda i: (0, 0)),
            pl.BlockSpec((tile, d_model), lambda i: (i % n_s, 0)),
        ],
        out_specs=pl.BlockSpec((tile, d_model), lambda i: (i, 0)),
        compiler_params=pltpu.CompilerParams(
            dimension_semantics=("parallel",)),
        cost_estimate=cost,
    )(ids, tcol, tab3d, ttab, pe)
    return out.reshape(B, S, d_model)


# speedup vs baseline: 1.4171x; 1.4171x over previous
import functools

import jax
import jax.numpy as jnp
from jax.experimental import pallas as pl
from jax.experimental.pallas import tpu as pltpu

_TOKEN_OFF = 4000
_TIME_ROWS = 64


def _gather_tile_kernel(TILE, ids_ref, tcol_ref, tab_ref, ttab_ref, pe_ref,
                        out_ref):
    base = pl.program_id(0) * TILE
    for mi in range(TILE):
        out_ref[mi] = tab_ref[ids_ref[base + mi], 0]
    lane = jax.lax.broadcasted_iota(jnp.int32, (TILE, _TIME_ROWS), 1)
    t_oh = (tcol_ref[...] == lane).astype(jnp.float32)
    tmm = jnp.dot(t_oh, ttab_ref[...], preferred_element_type=jnp.float32)
    out_ref[...] = out_ref[...] + (tmm + pe_ref[...])


def kernel(x, time, fused_table, pe_scaled):
    B, S = x.shape
    v_pad, d_model = fused_table.shape

    ids = x.astype(jnp.int32).reshape(B * S)
    tcol = time.astype(jnp.int32).reshape(B * S, 1)
    tab3d = fused_table.reshape(v_pad, 1, d_model)
    ttab = fused_table[_TOKEN_OFF:_TOKEN_OFF + _TIME_ROWS]
    pe = pe_scaled[:S]

    tile = 512
    while S % tile:
        tile //= 2
    n_s = S // tile
    grid = (B * n_s,)

    body = functools.partial(_gather_tile_kernel, tile)

    m_total = B * S
    bytes_accessed = (2 * m_total * 4
                      + v_pad * d_model * 4
                      + S * d_model * 4
                      + m_total * d_model * 4)
    cost = pl.CostEstimate(flops=3 * m_total * d_model, transcendentals=0,
                           bytes_accessed=bytes_accessed)

    out = pl.pallas_call(
        body,
        out_shape=jax.ShapeDtypeStruct((B * S, d_model), jnp.float32),
        grid=grid,
        in_specs=[
            pl.BlockSpec(memory_space=pltpu.SMEM),
            pl.BlockSpec((tile, 1), lambda i: (i, 0)),
            pl.BlockSpec((v_pad, 1, d_model), lambda i: (0, 0, 0)),
            pl.BlockSpec((_TIME_ROWS, d_model), lambda i: (0, 0)),
            pl.BlockSpec((tile, d_model), lambda i: (i % n_s, 0)),
        ],
        out_specs=pl.BlockSpec((tile, d_model), lambda i: (i, 0)),
        compiler_params=pltpu.CompilerParams(
            dimension_semantics=("parallel",)),
        cost_estimate=cost,
    )(ids, tcol, tab3d, ttab, pe)
    return out.reshape(B, S, d_model)


# speedup vs baseline: 1.8185x; 1.2832x over previous
import functools

import jax
import jax.numpy as jnp
from jax.experimental import pallas as pl
from jax.experimental.pallas import tpu as pltpu

_TOKEN_OFF = 4000
_TIME_ROWS = 64


def _gather_tile_kernel(TILE, ids_ref, tcol_ref, tab_ref, ttab_ref, pe_ref,
                        out_ref):
    base = pl.program_id(0) * TILE
    for mi in range(TILE):
        out_ref[mi] = tab_ref[ids_ref[base + mi], 0]
    lane = jax.lax.broadcasted_iota(jnp.int32, (TILE, _TIME_ROWS), 1)
    t_oh = (tcol_ref[...] == lane).astype(jnp.float32)
    tmm = jnp.dot(t_oh, ttab_ref[...], preferred_element_type=jnp.float32)
    out_ref[...] = out_ref[...] + (tmm + pe_ref[...])


def kernel(x, time, fused_table, pe_scaled):
    B, S = x.shape
    v_pad, d_model = fused_table.shape

    ids = x.astype(jnp.int32).reshape(B * S)
    tcol = time.astype(jnp.int32).reshape(B * S, 1)
    tab3d = fused_table.reshape(v_pad, 1, d_model)
    ttab = fused_table[_TOKEN_OFF:_TOKEN_OFF + _TIME_ROWS]
    pe = pe_scaled[:S]

    tile = 1024
    while S % tile:
        tile //= 2
    n_s = S // tile
    grid = (B * n_s,)

    body = functools.partial(_gather_tile_kernel, tile)

    m_total = B * S
    bytes_accessed = (2 * m_total * 4
                      + v_pad * d_model * 4
                      + S * d_model * 4
                      + m_total * d_model * 4)
    cost = pl.CostEstimate(flops=3 * m_total * d_model, transcendentals=0,
                           bytes_accessed=bytes_accessed)

    out = pl.pallas_call(
        body,
        out_shape=jax.ShapeDtypeStruct((B * S, d_model), jnp.float32),
        grid=grid,
        in_specs=[
            pl.BlockSpec(memory_space=pltpu.SMEM),
            pl.BlockSpec((tile, 1), lambda i: (i, 0)),
            pl.BlockSpec((v_pad, 1, d_model), lambda i: (0, 0, 0)),
            pl.BlockSpec((_TIME_ROWS, d_model), lambda i: (0, 0)),
            pl.BlockSpec((tile, d_model), lambda i: (i % n_s, 0)),
        ],
        out_specs=pl.BlockSpec((tile, d_model), lambda i: (i, 0)),
        compiler_params=pltpu.CompilerParams(
            dimension_semantics=("parallel",)),
        cost_estimate=cost,
    )(ids, tcol, tab3d, ttab, pe)
    return out.reshape(B, S, d_model)


# speedup vs baseline: 2.0802x; 1.1439x over previous
import functools

import jax
import jax.numpy as jnp
from jax.experimental import pallas as pl
from jax.experimental.pallas import tpu as pltpu

_TOKEN_OFF = 4000
_TIME_ROWS = 64


def _gather_tile_kernel(TILE, ids_ref, tcol_ref, tab_ref, ttab_ref, pe_ref,
                        out_ref):
    base = pl.program_id(0) * TILE
    for mi in range(TILE):
        out_ref[mi] = tab_ref[ids_ref[base + mi], 0]
    lane = jax.lax.broadcasted_iota(jnp.int32, (TILE, _TIME_ROWS), 1)
    t_oh = (tcol_ref[...] == lane).astype(jnp.float32)
    tmm = jnp.dot(t_oh, ttab_ref[...], preferred_element_type=jnp.float32)
    out_ref[...] = out_ref[...] + (tmm + pe_ref[...])


def kernel(x, time, fused_table, pe_scaled):
    B, S = x.shape
    v_pad, d_model = fused_table.shape

    ids = x.astype(jnp.int32).reshape(B * S)
    tcol = time.astype(jnp.int32).reshape(B * S, 1)
    tab3d = fused_table.reshape(v_pad, 1, d_model)
    ttab = fused_table[_TOKEN_OFF:_TOKEN_OFF + _TIME_ROWS]
    pe = pe_scaled[:S]

    tile = 2048
    while S % tile:
        tile //= 2
    n_s = S // tile
    grid = (B * n_s,)

    body = functools.partial(_gather_tile_kernel, tile)

    m_total = B * S
    bytes_accessed = (2 * m_total * 4
                      + v_pad * d_model * 4
                      + S * d_model * 4
                      + m_total * d_model * 4)
    cost = pl.CostEstimate(flops=3 * m_total * d_model, transcendentals=0,
                           bytes_accessed=bytes_accessed)

    out = pl.pallas_call(
        body,
        out_shape=jax.ShapeDtypeStruct((B * S, d_model), jnp.float32),
        grid=grid,
        in_specs=[
            pl.BlockSpec(memory_space=pltpu.SMEM),
            pl.BlockSpec((tile, 1), lambda i: (i, 0)),
            pl.BlockSpec((v_pad, 1, d_model), lambda i: (0, 0, 0)),
            pl.BlockSpec((_TIME_ROWS, d_model), lambda i: (0, 0)),
            pl.BlockSpec((tile, d_model), lambda i: (i % n_s, 0)),
        ],
        out_specs=pl.BlockSpec((tile, d_model), lambda i: (i, 0)),
        compiler_params=pltpu.CompilerParams(
            dimension_semantics=("parallel",)),
        cost_estimate=cost,
    )(ids, tcol, tab3d, ttab, pe)
    return out.reshape(B, S, d_model)


# speedup vs baseline: 2.1015x; 1.0103x over previous
import functools

import jax
import jax.numpy as jnp
from jax.experimental import pallas as pl
from jax.experimental.pallas import tpu as pltpu

_TOKEN_OFF = 4000
_TIME_ROWS = 64


def _gather_tile_kernel(TILE, ids_ref, tcol_ref, tab_ref, ttab_ref, pe_ref,
                        out_ref):
    base = pl.program_id(0) * TILE
    for mi in range(TILE):
        out_ref[mi] = tab_ref[ids_ref[base + mi], 0]
    lane = jax.lax.broadcasted_iota(jnp.int32, (TILE, _TIME_ROWS), 1)
    t_oh = (tcol_ref[...] == lane).astype(jnp.float32)
    tmm = jnp.dot(t_oh, ttab_ref[...], preferred_element_type=jnp.float32)
    out_ref[...] = out_ref[...] + (tmm + pe_ref[...])


def kernel(x, time, fused_table, pe_scaled):
    B, S = x.shape
    v_pad, d_model = fused_table.shape

    ids = x.astype(jnp.int32).reshape(B * S)
    tcol = time.astype(jnp.int32).reshape(B * S, 1)
    tab3d = fused_table.reshape(v_pad, 1, d_model)
    ttab = fused_table[_TOKEN_OFF:_TOKEN_OFF + _TIME_ROWS]
    pe = pe_scaled[:S]

    tile = 4096
    while S % tile:
        tile //= 2
    n_s = S // tile
    grid = (B * n_s,)

    body = functools.partial(_gather_tile_kernel, tile)

    m_total = B * S
    bytes_accessed = (2 * m_total * 4
                      + v_pad * d_model * 4
                      + S * d_model * 4
                      + m_total * d_model * 4)
    cost = pl.CostEstimate(flops=3 * m_total * d_model, transcendentals=0,
                           bytes_accessed=bytes_accessed)

    out = pl.pallas_call(
        body,
        out_shape=jax.ShapeDtypeStruct((B * S, d_model), jnp.float32),
        grid=grid,
        in_specs=[
            pl.BlockSpec(memory_space=pltpu.SMEM),
            pl.BlockSpec((tile, 1), lambda i: (i, 0)),
            pl.BlockSpec((v_pad, 1, d_model), lambda i: (0, 0, 0)),
            pl.BlockSpec((_TIME_ROWS, d_model), lambda i: (0, 0)),
            pl.BlockSpec((tile, d_model), lambda i: (i % n_s, 0)),
        ],
        out_specs=pl.BlockSpec((tile, d_model), lambda i: (i, 0)),
        compiler_params=pltpu.CompilerParams(
            dimension_semantics=("parallel",)),
        cost_estimate=cost,
    )(ids, tcol, tab3d, ttab, pe)
    return out.reshape(B, S, d_model)
